# baseline (speedup 1.0000x reference)
import functools

import jax
import jax.numpy as jnp
from jax.experimental import pallas as pl
from jax.experimental.pallas import tpu as pltpu

_NBUF = 3


def _gcn_body(x_ref, adj_hbm, w1_ref, b1_ref, w2_ref, b2_ref,
              feat_hbm, out_hbm,
              abuf, fbuf, obuf, support_ref,
              in_sems, f_sems, o_sems, *, n_chunks, bn):

    def adj_cp(k):
        return pltpu.make_async_copy(
            adj_hbm.at[pl.ds(k * bn, bn), :], abuf.at[k % _NBUF],
            in_sems.at[k % _NBUF])

    def feat_cp(k):
        return pltpu.make_async_copy(
            fbuf.at[k % 2], feat_hbm.at[pl.ds(k * bn, bn), :], f_sems.at[k % 2])

    def out_cp(k):
        return pltpu.make_async_copy(
            obuf.at[k % 2], out_hbm.at[pl.ds(k * bn, bn), :], o_sems.at[k % 2])

    for k in range(min(_NBUF, n_chunks)):
        adj_cp(k).start()

    support_ref[...] = (
        jnp.dot(x_ref[...].astype(jnp.bfloat16),
                w1_ref[...].astype(jnp.bfloat16),
                preferred_element_type=jnp.float32)
        + b1_ref[...]
    ).astype(jnp.bfloat16)

    for k in range(n_chunks):
        adj_cp(k).wait()
        h = jnp.dot(abuf[k % _NBUF].astype(jnp.bfloat16), support_ref[...],
                    preferred_element_type=jnp.float32)
        if k + _NBUF < n_chunks:
            adj_cp(k + _NBUF).start()
        feat = jnp.maximum(h, 0.0)
        if k >= 2:
            feat_cp(k - 2).wait()
            out_cp(k - 2).wait()
        fbuf[k % 2] = feat
        obuf[k % 2] = jax.nn.sigmoid(
            jnp.dot(feat.astype(jnp.bfloat16), w2_ref[...].astype(jnp.bfloat16),
                    preferred_element_type=jnp.float32)
            + b2_ref[...]
        )
        feat_cp(k).start()
        out_cp(k).start()

    for k in range(max(0, n_chunks - 2), n_chunks):
        feat_cp(k).wait()
        out_cp(k).wait()


@functools.partial(jax.jit, static_argnames=("bn",))
def _gcn_fused(x, adj, W1, b1, W2, b2, bn=512):
    n, f = x.shape
    h_dim = W1.shape[1]
    c = W2.shape[1]
    n_chunks = n // bn
    b1r = b1.reshape(1, h_dim)
    b2r = b2.reshape(1, c)
    feature, out = pl.pallas_call(
        functools.partial(_gcn_body, n_chunks=n_chunks, bn=bn),
        in_specs=[
            pl.BlockSpec(memory_space=pltpu.MemorySpace.VMEM),
            pl.BlockSpec(memory_space=pltpu.MemorySpace.HBM),
            pl.BlockSpec(memory_space=pltpu.MemorySpace.VMEM),
            pl.BlockSpec(memory_space=pltpu.MemorySpace.VMEM),
            pl.BlockSpec(memory_space=pltpu.MemorySpace.VMEM),
            pl.BlockSpec(memory_space=pltpu.MemorySpace.VMEM),
        ],
        out_specs=[
            pl.BlockSpec(memory_space=pltpu.MemorySpace.HBM),
            pl.BlockSpec(memory_space=pltpu.MemorySpace.HBM),
        ],
        out_shape=[
            jax.ShapeDtypeStruct((n, h_dim), jnp.float32),
            jax.ShapeDtypeStruct((n, c), jnp.float32),
        ],
        scratch_shapes=[
            pltpu.VMEM((_NBUF, bn, n), jnp.float32),
            pltpu.VMEM((2, bn, h_dim), jnp.float32),
            pltpu.VMEM((2, bn, c), jnp.float32),
            pltpu.VMEM((n, h_dim), jnp.bfloat16),
            pltpu.SemaphoreType.DMA((_NBUF,)),
            pltpu.SemaphoreType.DMA((2,)),
            pltpu.SemaphoreType.DMA((2,)),
        ],
    )(x, adj, W1, b1r, W2, b2r)
    return feature, out


def kernel(x, adj, W1, b1, W2, b2):
    return _gcn_fused(x, adj, W1, b1, W2, b2)
